# baseline (speedup 1.0000x reference)
import functools

import jax
import jax.numpy as jnp
from jax import lax
from jax.experimental import pallas as pl
from jax.experimental.pallas import tpu as pltpu
from jax.experimental.pallas import tpu_sc as plsc

D = 64
DP = 128
COND = 256
B = 4096
H = 20
V = 1_000_000

NC = 2
NS = 16
L = 16
NW = NC * NS
BPW = B // NW
IPW = BPW * H
SL = 128
NSL = IPW // SL
NBUF = 2

TC = 8192
TH = TC // 2


def _tc_stage(tableT):
    grid = (V + TC - 1) // TC

    def tr(x_ref, i_ref, o_ref):
        dn = (((0,), (0,)), ((), ()))
        o_ref[:, pl.ds(0, D)] = jax.lax.dot_general(
            x_ref[:, pl.ds(0, TH)], i_ref[...], dn,
            preferred_element_type=jnp.float32,
        )
        o_ref[:, pl.ds(D, D)] = jax.lax.dot_general(
            x_ref[:, pl.ds(TH, TH)], i_ref[...], dn,
            preferred_element_type=jnp.float32,
        )

    return pl.pallas_call(
        tr,
        grid=(grid,),
        in_specs=[
            pl.BlockSpec((D, TC), lambda i: (0, i)),
            pl.BlockSpec((D, D), lambda i: (0, 0)),
        ],
        out_specs=pl.BlockSpec((TH, DP), lambda i: (i, 0)),
        out_shape=jax.ShapeDtypeStruct((grid * TH, DP), jnp.float32),
    )(tableT, jnp.eye(D, dtype=jnp.float32))


def _sc_pool(ids3d, seg3d, table3):
    mesh = plsc.VectorSubcoreMesh(core_axis_name="c", subcore_axis_name="s")

    @functools.partial(
        pl.kernel,
        mesh=mesh,
        out_type=jax.ShapeDtypeStruct((B, D), jnp.float32),
        scratch_types=[
            pltpu.VMEM((NSL, SL), jnp.int32),
            pltpu.VMEM((NSL, SL), jnp.int32),
            pltpu.VMEM((NBUF, SL, DP), jnp.float32),
            pltpu.VMEM_SHARED((NS, 2 * BPW, DP), jnp.float32),
            pltpu.VMEM((BPW, D), jnp.float32),
            pltpu.SemaphoreType.DMA,
            pltpu.SemaphoreType.DMA,
        ],
    )
    def k(ids_hbm, seg_hbm, table_hbm, out_hbm, ids_v, seg_v, buf_v, acc_sh,
          pool_v, sem_g, sem_s):
        cid = lax.axis_index("c")
        sid = lax.axis_index("s")
        wid = sid * NC + cid
        base = wid * BPW
        acc_v = acc_sh.at[sid]
        pltpu.sync_copy(ids_hbm.at[wid], ids_v)
        pltpu.sync_copy(seg_hbm.at[wid], seg_v)

        def zero_body(i, _):
            z = jnp.zeros((L,), jnp.float32)
            for cc in range(DP // L):
                buf_v[0, i, pl.ds(cc * L, L)] = z
            return 0

        lax.fori_loop(0, SL, zero_body, 0)
        pltpu.sync_copy(buf_v.at[0], acc_v.at[pl.ds(0, SL), :])
        pltpu.sync_copy(buf_v.at[0], acc_v.at[pl.ds(SL, SL), :])

        def gather(s):
            return pltpu.async_copy(
                table_hbm.at[ids_v.at[s]], buf_v.at[s % NBUF], sem_g
            )

        def scat(s):
            return pltpu.async_copy(
                buf_v.at[s % NBUF], acc_v.at[seg_v.at[s]], sem_s, add=True
            )

        gcps = [gather(0)]
        for s in range(NSL):
            gcps[s].wait()
            if s + 1 < NSL:
                gcps.append(gather(s + 1))
            scat(s).wait()

        pltpu.sync_copy(acc_v.at[pl.ds(0, SL), :], buf_v.at[0])
        pltpu.sync_copy(acc_v.at[pl.ds(SL, SL), :], buf_v.at[1])

        def comb_body(i, _):
            for cc in range(D // L):
                pool_v[i, pl.ds(cc * L, L)] = (
                    buf_v[0, i, pl.ds(cc * L, L)]
                    + buf_v[1, i, pl.ds(D + cc * L, L)]
                )
            return 0

        lax.fori_loop(0, BPW, comb_body, 0)
        pltpu.sync_copy(pool_v, out_hbm.at[pl.ds(base, BPW)])

    return k(ids3d, seg3d, table3)


def _tc_proj(pooled, W, b2):
    BM = 512

    def mm(p_ref, w_ref, b_ref, o_ref):
        o_ref[...] = (
            jnp.dot(
                p_ref[...] * (1.0 / H), w_ref[...],
                preferred_element_type=jnp.float32,
            )
            + b_ref[...]
        )

    return pl.pallas_call(
        mm,
        grid=(B // BM,),
        in_specs=[
            pl.BlockSpec((BM, D), lambda i: (i, 0)),
            pl.BlockSpec((D, COND), lambda i: (0, 0)),
            pl.BlockSpec((1, COND), lambda i: (0, 0)),
        ],
        out_specs=pl.BlockSpec((BM, COND), lambda i: (i, 0)),
        out_shape=jax.ShapeDtypeStruct((B, COND), jnp.float32),
    )(pooled, W, b2)


def kernel(token_ids, table, W, b):
    ids = token_ids.astype(jnp.int32).reshape(B * H)
    vrow = (ids >> 13) * TH + (ids & (TH - 1))
    half = (ids >> 12) & 1
    ids3d = vrow.reshape(NW, NSL, SL)
    b_local = (jnp.arange(B * H, dtype=jnp.int32) // H) % BPW
    seg3d = (b_local + BPW * half).reshape(NW, NSL, SL)
    table3 = _tc_stage(table.T)
    pooled = _sc_pool(ids3d, seg3d, table3)
    return _tc_proj(pooled, W, b.reshape(1, COND))


# speedup vs baseline: 1.1156x; 1.1156x over previous
import functools

import jax
import jax.numpy as jnp
from jax import lax
from jax.experimental import pallas as pl
from jax.experimental.pallas import tpu as pltpu
from jax.experimental.pallas import tpu_sc as plsc

D = 64
DP = 128
COND = 256
B = 4096
H = 20
V = 1_000_000

NC = 2
NS = 16
L = 16
NW = NC * NS
BPW = B // NW
IPW = BPW * H
SL = 128
NSL = IPW // SL
NBUF = 2

TC = 16384
TH = TC // 2


def _tc_stage(tableT):
    grid = (V + TC - 1) // TC

    def tr(x_ref, o_ref):
        o_ref[:, pl.ds(0, D)] = x_ref[:, pl.ds(0, TH)].T
        o_ref[:, pl.ds(D, D)] = x_ref[:, pl.ds(TH, TH)].T

    return pl.pallas_call(
        tr,
        grid=(grid,),
        in_specs=[pl.BlockSpec((D, TC), lambda i: (0, i))],
        out_specs=pl.BlockSpec((TH, DP), lambda i: (i, 0)),
        out_shape=jax.ShapeDtypeStruct((grid * TH, DP), jnp.float32),
    )(tableT)


def _sc_pool(ids3d, seg3d, table3):
    mesh = plsc.VectorSubcoreMesh(core_axis_name="c", subcore_axis_name="s")

    @functools.partial(
        pl.kernel,
        mesh=mesh,
        out_type=jax.ShapeDtypeStruct((B, D), jnp.float32),
        scratch_types=[
            pltpu.VMEM((NSL, SL), jnp.int32),
            pltpu.VMEM((NSL, SL), jnp.int32),
            pltpu.VMEM((NBUF, SL, DP), jnp.float32),
            pltpu.VMEM_SHARED((NS, 2 * BPW, DP), jnp.float32),
            pltpu.VMEM((BPW, D), jnp.float32),
            pltpu.SemaphoreType.DMA,
            pltpu.SemaphoreType.DMA,
        ],
    )
    def k(ids_hbm, seg_hbm, table_hbm, out_hbm, ids_v, seg_v, buf_v, acc_sh,
          pool_v, sem_g, sem_s):
        cid = lax.axis_index("c")
        sid = lax.axis_index("s")
        wid = sid * NC + cid
        base = wid * BPW
        acc_v = acc_sh.at[sid]
        pltpu.sync_copy(ids_hbm.at[wid], ids_v)
        pltpu.sync_copy(seg_hbm.at[wid], seg_v)

        def zero_body(i, _):
            z = jnp.zeros((L,), jnp.float32)
            for cc in range(DP // L):
                buf_v[0, i, pl.ds(cc * L, L)] = z
            return 0

        lax.fori_loop(0, SL, zero_body, 0)
        pltpu.sync_copy(buf_v.at[0], acc_v.at[pl.ds(0, SL), :])
        pltpu.sync_copy(buf_v.at[0], acc_v.at[pl.ds(SL, SL), :])

        def gather(s):
            return pltpu.async_copy(
                table_hbm.at[ids_v.at[s]], buf_v.at[s % NBUF], sem_g
            )

        def scat(s):
            return pltpu.async_copy(
                buf_v.at[s % NBUF], acc_v.at[seg_v.at[s]], sem_s, add=True
            )

        gcps = [gather(0)]
        for s in range(NSL):
            gcps[s].wait()
            if s + 1 < NSL:
                gcps.append(gather(s + 1))
            scat(s).wait()

        pltpu.sync_copy(acc_v.at[pl.ds(0, SL), :], buf_v.at[0])
        pltpu.sync_copy(acc_v.at[pl.ds(SL, SL), :], buf_v.at[1])

        def comb_body(i, _):
            for cc in range(D // L):
                pool_v[i, pl.ds(cc * L, L)] = (
                    buf_v[0, i, pl.ds(cc * L, L)]
                    + buf_v[1, i, pl.ds(D + cc * L, L)]
                )
            return 0

        lax.fori_loop(0, BPW, comb_body, 0)
        pltpu.sync_copy(pool_v, out_hbm.at[pl.ds(base, BPW)])

    return k(ids3d, seg3d, table3)


def _tc_proj(pooled, W, b2):
    BM = 512

    def mm(p_ref, w_ref, b_ref, o_ref):
        o_ref[...] = (
            jnp.dot(
                p_ref[...] * (1.0 / H), w_ref[...],
                preferred_element_type=jnp.float32,
            )
            + b_ref[...]
        )

    return pl.pallas_call(
        mm,
        grid=(B // BM,),
        in_specs=[
            pl.BlockSpec((BM, D), lambda i: (i, 0)),
            pl.BlockSpec((D, COND), lambda i: (0, 0)),
            pl.BlockSpec((1, COND), lambda i: (0, 0)),
        ],
        out_specs=pl.BlockSpec((BM, COND), lambda i: (i, 0)),
        out_shape=jax.ShapeDtypeStruct((B, COND), jnp.float32),
    )(pooled, W, b2)


def kernel(token_ids, table, W, b):
    ids = token_ids.astype(jnp.int32).reshape(B * H)
    tc_bits = TC.bit_length() - 1
    vrow = (ids >> tc_bits) * TH + (ids & (TH - 1))
    half = (ids >> (tc_bits - 1)) & 1
    ids3d = vrow.reshape(NW, NSL, SL)
    b_local = (jnp.arange(B * H, dtype=jnp.int32) // H) % BPW
    seg3d = (b_local + BPW * half).reshape(NW, NSL, SL)
    table3 = _tc_stage(table.T)
    pooled = _sc_pool(ids3d, seg3d, table3)
    return _tc_proj(pooled, W, b.reshape(1, COND))


# speedup vs baseline: 1.1735x; 1.0519x over previous
import functools

import jax
import jax.numpy as jnp
from jax import lax
from jax.experimental import pallas as pl
from jax.experimental.pallas import tpu as pltpu
from jax.experimental.pallas import tpu_sc as plsc

D = 64
DP = 128
COND = 256
B = 4096
H = 20
V = 1_000_000

NC = 2
NS = 16
L = 16
NW = NC * NS
BPW = B // NW
IPW = BPW * H
SL = 128
NSL = IPW // SL
NBUF = 2

TC = 32768
TH = TC // 2


def _tc_stage(tableT):
    grid = (V + TC - 1) // TC

    def tr(x_ref, o_ref):
        o_ref[:, pl.ds(0, D)] = x_ref[:, pl.ds(0, TH)].T
        o_ref[:, pl.ds(D, D)] = x_ref[:, pl.ds(TH, TH)].T

    return pl.pallas_call(
        tr,
        grid=(grid,),
        in_specs=[pl.BlockSpec((D, TC), lambda i: (0, i))],
        out_specs=pl.BlockSpec((TH, DP), lambda i: (i, 0)),
        out_shape=jax.ShapeDtypeStruct((grid * TH, DP), jnp.float32),
    )(tableT)


def _sc_pool(ids3d, seg3d, table3):
    mesh = plsc.VectorSubcoreMesh(core_axis_name="c", subcore_axis_name="s")

    @functools.partial(
        pl.kernel,
        mesh=mesh,
        out_type=jax.ShapeDtypeStruct((B, D), jnp.float32),
        scratch_types=[
            pltpu.VMEM((NSL, SL), jnp.int32),
            pltpu.VMEM((NSL, SL), jnp.int32),
            pltpu.VMEM((NBUF, SL, DP), jnp.float32),
            pltpu.VMEM_SHARED((NS, 2 * BPW, DP), jnp.float32),
            pltpu.VMEM((BPW, D), jnp.float32),
            pltpu.SemaphoreType.DMA,
            pltpu.SemaphoreType.DMA,
        ],
    )
    def k(ids_hbm, seg_hbm, table_hbm, out_hbm, ids_v, seg_v, buf_v, acc_sh,
          pool_v, sem_g, sem_s):
        cid = lax.axis_index("c")
        sid = lax.axis_index("s")
        wid = sid * NC + cid
        base = wid * BPW
        acc_v = acc_sh.at[sid]
        pltpu.sync_copy(ids_hbm.at[wid], ids_v)
        pltpu.sync_copy(seg_hbm.at[wid], seg_v)

        def zero_body(i, _):
            z = jnp.zeros((L,), jnp.float32)
            for cc in range(DP // L):
                buf_v[0, i, pl.ds(cc * L, L)] = z
            return 0

        lax.fori_loop(0, SL, zero_body, 0)
        pltpu.sync_copy(buf_v.at[0], acc_v.at[pl.ds(0, SL), :])
        pltpu.sync_copy(buf_v.at[0], acc_v.at[pl.ds(SL, SL), :])

        def gather(s):
            return pltpu.async_copy(
                table_hbm.at[ids_v.at[s]], buf_v.at[s % NBUF], sem_g
            )

        def scat(s):
            return pltpu.async_copy(
                buf_v.at[s % NBUF], acc_v.at[seg_v.at[s]], sem_s, add=True
            )

        gcps = [gather(0)]
        for s in range(NSL):
            gcps[s].wait()
            if s + 1 < NSL:
                gcps.append(gather(s + 1))
            scat(s).wait()

        pltpu.sync_copy(acc_v.at[pl.ds(0, SL), :], buf_v.at[0])
        pltpu.sync_copy(acc_v.at[pl.ds(SL, SL), :], buf_v.at[1])

        def comb_body(i, _):
            for cc in range(D // L):
                pool_v[i, pl.ds(cc * L, L)] = (
                    buf_v[0, i, pl.ds(cc * L, L)]
                    + buf_v[1, i, pl.ds(D + cc * L, L)]
                )
            return 0

        lax.fori_loop(0, BPW, comb_body, 0)
        pltpu.sync_copy(pool_v, out_hbm.at[pl.ds(base, BPW)])

    return k(ids3d, seg3d, table3)


def _tc_proj(pooled, W, b2):
    BM = 512

    def mm(p_ref, w_ref, b_ref, o_ref):
        o_ref[...] = (
            jnp.dot(
                p_ref[...] * (1.0 / H), w_ref[...],
                preferred_element_type=jnp.float32,
            )
            + b_ref[...]
        )

    return pl.pallas_call(
        mm,
        grid=(B // BM,),
        in_specs=[
            pl.BlockSpec((BM, D), lambda i: (i, 0)),
            pl.BlockSpec((D, COND), lambda i: (0, 0)),
            pl.BlockSpec((1, COND), lambda i: (0, 0)),
        ],
        out_specs=pl.BlockSpec((BM, COND), lambda i: (i, 0)),
        out_shape=jax.ShapeDtypeStruct((B, COND), jnp.float32),
    )(pooled, W, b2)


def kernel(token_ids, table, W, b):
    ids = token_ids.astype(jnp.int32).reshape(B * H)
    tc_bits = TC.bit_length() - 1
    vrow = (ids >> tc_bits) * TH + (ids & (TH - 1))
    half = (ids >> (tc_bits - 1)) & 1
    ids3d = vrow.reshape(NW, NSL, SL)
    b_local = (jnp.arange(B * H, dtype=jnp.int32) // H) % BPW
    seg3d = (b_local + BPW * half).reshape(NW, NSL, SL)
    table3 = _tc_stage(table.T)
    pooled = _sc_pool(ids3d, seg3d, table3)
    return _tc_proj(pooled, W, b.reshape(1, COND))
